# Initial kernel scaffold; baseline (speedup 1.0000x reference)
import jax
import jax.numpy as jnp
from jax.experimental import pallas as pl


def kernel(input, changeIndexes, outputState):
    raise NotImplementedError("write your pallas kernel here")


# speedup vs baseline: 3.3034x; 3.3034x over previous
import functools

import jax
import jax.numpy as jnp
from jax import lax
from jax.experimental import pallas as pl
from jax.experimental.pallas import tpu as pltpu
from jax.experimental.pallas import tpu_sc as plsc

_N_IDX = 32768
_MASK_N = 65536
_NC = 2
_NS = 16
_NW = _NC * _NS
_SLICE = _MASK_N // _NW
_LANES = 16


def _mask_body(idx_hbm, out_hbm, idx_v, slice_v):
    wid = lax.axis_index("s") * _NC + lax.axis_index("c")
    base = wid * _SLICE

    zero = jnp.zeros((_LANES,), jnp.int32)

    def zero_body(i, carry):
        slice_v[pl.ds(i * _LANES, _LANES)] = zero
        return carry

    lax.fori_loop(0, _SLICE // _LANES, zero_body, 0)

    pltpu.sync_copy(idx_hbm, idx_v)

    ones = jnp.ones((_LANES,), jnp.int32)

    def body(i, carry):
        idx = idx_v[pl.ds(i * _LANES, _LANES)]
        local = idx - base
        m = (local >= 0) & (local < _SLICE)
        safe = jnp.where(m, local, 0)
        plsc.store_scatter(slice_v, [safe], ones, mask=m)
        return carry

    lax.fori_loop(0, _N_IDX // _LANES, body, 0)

    pltpu.sync_copy(slice_v, out_hbm.at[pl.ds(base, _SLICE)])


def _make_mask(change_indexes):
    mesh = plsc.VectorSubcoreMesh(core_axis_name="c", subcore_axis_name="s")
    k = functools.partial(
        pl.kernel,
        mesh=mesh,
        out_type=jax.ShapeDtypeStruct((_MASK_N,), jnp.int32),
        scratch_types=[
            pltpu.VMEM((_N_IDX,), jnp.int32),
            pltpu.VMEM((_SLICE,), jnp.int32),
        ],
        compiler_params=pltpu.CompilerParams(needs_layout_passes=False),
    )(_mask_body)
    return k(change_indexes)


_CB = 8


def _pool_body(mask_ref, x_ref, state_ref, out_ref, xr_ref):
    for wq in range(4):
        top = x_ref[:, :, wq::8, :][0]
        bot = x_ref[:, :, wq + 4::8, :][0]
        xr_ref[wq] = jnp.maximum(top, bot)
    row = jax.lax.broadcasted_iota(jnp.int32, (128, 64), 0)
    col = jax.lax.broadcasted_iota(jnp.int32, (128, 64), 1)
    sel = (row == 2 * col).astype(jnp.float32)
    cb = xr_ref.shape[1]
    chunks = []
    for wq in range(4):
        m = xr_ref[wq]
        sh = jnp.concatenate([m[:, :, 1:], m[:, :, :1]], axis=-1)
        m2 = jnp.maximum(m, sh).reshape(cb * 256, 128)
        c = jax.lax.dot_general(m2, sel, (((1,), (0,)), ((), ())),
                                preferred_element_type=jnp.float32)
        chunks.append(c.reshape(cb, 256, 64))
    pooled = jnp.concatenate(chunks, axis=-1)
    m = mask_ref[...] > 0
    out_ref[0] = jnp.where(m, pooled, state_ref[0])


def _pool_select(x, state, mask):
    n, c, h, w = x.shape
    oh, ow = h // 2, w // 2
    x2 = x.reshape(n, c, (h * w) // 128, 128)
    return pl.pallas_call(
        _pool_body,
        grid=(c // _CB,),
        in_specs=[
            pl.BlockSpec((1, oh, ow), lambda i: (0, 0, 0)),
            pl.BlockSpec((1, _CB, (h * w) // 128, 128), lambda i: (0, i, 0, 0)),
            pl.BlockSpec((1, _CB, oh, ow), lambda i: (0, i, 0, 0)),
        ],
        out_specs=pl.BlockSpec((1, _CB, oh, ow), lambda i: (0, i, 0, 0)),
        out_shape=jax.ShapeDtypeStruct((n, c, oh, ow), x.dtype),
        scratch_shapes=[pltpu.VMEM((4, _CB, oh, 128), x.dtype)],
    )(mask, x2, state)


def kernel(input, changeIndexes, outputState):
    n, c, h, w = input.shape
    oh, ow = h // 2, w // 2
    mask = _make_mask(changeIndexes).reshape(1, oh, ow)
    return _pool_select(input, outputState, mask)
